# baseline (speedup 1.0000x reference)
import functools

import jax
import jax.numpy as jnp
from jax import lax
from jax.experimental import pallas as pl
from jax.experimental.pallas import tpu as pltpu
from jax.experimental.pallas import tpu_sc as plsc

K = 64
NWORKERS = 32


def _encode_body(xb, we, be, bd, out):
    j = pl.program_id(1)
    C = we.shape[0]
    xc = xb[...] - bd[...]
    acc = lax.dot_general(xc, we[...], (((1,), (1,)), ((), ())),
                          preferred_element_type=jnp.float32)
    out[...] = jnp.maximum(acc + be[0, pl.ds(j * C, C)][None, :], 0.0)


def _threshold_body(rows_per_w, post_hbm, t_hbm,
                    rowbuf, hist, cnts, sufs, cand, tout, sem):
    c = lax.axis_index("c")
    s = lax.axis_index("s")
    wid = s * 2 + c
    base = wid * rows_per_w
    lanes = lax.iota(jnp.int32, 16)
    ones_f = jnp.ones((16,), jnp.float32)
    zeros16 = jnp.zeros((16,), jnp.int32)
    zeros16_f = jnp.zeros((16,), jnp.float32)

    def row_step(r, carry):
        pltpu.sync_copy(post_hbm.at[base + r], rowbuf)

        def zero_body(i, _):
            hist[pl.ds(i * 16, 16)] = zeros16_f
            return 0
        lax.fori_loop(0, 256, zero_body, 0)

        def hist_body(i, _):
            x = rowbuf[pl.ds(i * 16, 16)]
            b = jnp.maximum(lax.bitcast_convert_type(x, jnp.int32) >> 23, 0)
            plsc.addupdate_scatter(hist, [lanes * 256 + b], ones_f,
                                   mask=jnp.full((16,), True))
            return 0
        lax.fori_loop(0, 1024, hist_body, 0, unroll=8)

        def merge_body(i, _):
            acc = zeros16_f
            for l in range(16):
                acc = acc + hist[pl.ds(l * 256 + i * 16, 16)]
            cnts[pl.ds(i * 16, 16)] = acc
            return 0
        lax.fori_loop(0, 16, merge_body, 0)

        def scan_body(k, carry):
            carrysum, bstar = carry
            i = 15 - k
            cvec = cnts[pl.ds(i * 16, 16)]
            cs = plsc.cumsum(cvec)
            tot = jnp.sum(cvec)
            suffix = (carrysum + tot) - cs + cvec
            sufs[pl.ds(i * 16, 16)] = suffix
            bidx = i * 16 + lanes
            bstar = jnp.maximum(bstar,
                                jnp.where(suffix >= float(K), bidx, -1))
            return (carrysum + tot, bstar)
        _, bstar_vec = lax.fori_loop(0, 16, scan_body,
                                     (jnp.float32(0), jnp.full((16,), -1,
                                                               jnp.int32)))
        bs = jnp.max(bstar_vec)
        bs_v = jnp.full((16,), 0, jnp.int32) + bs
        cnt_b = jnp.max(plsc.load_gather(cnts, [bs_v]))
        suf_b = jnp.max(plsc.load_gather(sufs, [bs_v]))
        k2 = (jnp.float32(K) - (suf_b - cnt_b)).astype(jnp.int32)

        def compact_body(i, off):
            x = rowbuf[pl.ds(i * 16, 16)]
            bits = lax.bitcast_convert_type(x, jnp.int32)
            b = jnp.maximum(bits >> 23, 0)
            m = b == bs
            pos = plsc.cumsum(m.astype(jnp.int32))
            plsc.store_scatter(cand, [off + pos - 1], bits, mask=m)
            return off + jnp.max(pos)
        ncand = lax.fori_loop(0, 1024, compact_body, jnp.int32(0))
        nv = (ncand + 15) // 16

        def bit_step(bi, t):
            candt = t | (jnp.int32(1) << (22 - bi))

            def inner(v, accv):
                cb = cand[pl.ds(v * 16, 16)]
                valid = (v * 16 + lanes) < ncand
                ok = jnp.logical_and(cb >= candt, valid)
                return accv + jnp.where(ok, 1, 0)
            accv = lax.fori_loop(0, nv, inner, zeros16)
            return jnp.where(jnp.sum(accv) >= k2, candt, t)
        t = lax.fori_loop(0, 23, bit_step, bs << 23)

        tv = lax.bitcast_convert_type(zeros16 + t, jnp.float32)
        plsc.store_scatter(tout, [zeros16 + r], tv, mask=lanes == 0)
        return carry

    lax.fori_loop(0, rows_per_w, row_step, 0)
    pltpu.sync_copy(tout, t_hbm.at[pl.ds(base, rows_per_w)])


def _decode_body(xe, wd, tb, bd, out):
    j = pl.program_id(1)

    @pl.when(j == 0)
    def _():
        out[...] = jnp.broadcast_to(bd[...], out.shape)

    v = xe[...]
    m = jnp.where(v >= tb[...], v, 0.0)
    out[...] += lax.dot_general(m, wd[...], (((1,), (1,)), ((), ())),
                                preferred_element_type=jnp.float32)


def kernel(x, W_enc, b_enc, W_dec, b_dec):
    N, D = x.shape
    S = W_enc.shape[0]
    RE = min(1024, N)
    CE = min(1024, S)
    RD = min(512, N)
    CD = min(2048, S)

    post = pl.pallas_call(
        _encode_body,
        grid=(N // RE, S // CE),
        in_specs=[
            pl.BlockSpec((RE, D), lambda i, j: (i, 0)),
            pl.BlockSpec((CE, D), lambda i, j: (j, 0)),
            pl.BlockSpec((1, S), lambda i, j: (0, 0)),
            pl.BlockSpec((1, D), lambda i, j: (0, 0)),
        ],
        out_specs=pl.BlockSpec((RE, CE), lambda i, j: (i, j)),
        out_shape=jax.ShapeDtypeStruct((N, S), jnp.float32),
    )(x, W_enc, b_enc.reshape(1, S), b_dec.reshape(1, D))

    rows_per_w = N // NWORKERS
    mesh = plsc.VectorSubcoreMesh(core_axis_name="c", subcore_axis_name="s",
                                  num_cores=2, num_subcores=16)
    t = pl.kernel(
        functools.partial(_threshold_body, rows_per_w),
        out_type=jax.ShapeDtypeStruct((N,), jnp.float32),
        mesh=mesh,
        compiler_params=pltpu.CompilerParams(needs_layout_passes=False),
        scratch_types=[
            pltpu.VMEM((S,), jnp.float32),
            pltpu.VMEM((4096,), jnp.float32),
            pltpu.VMEM((256,), jnp.float32),
            pltpu.VMEM((256,), jnp.float32),
            pltpu.VMEM((S,), jnp.int32),
            pltpu.VMEM((rows_per_w,), jnp.float32),
            pltpu.SemaphoreType.DMA,
        ],
    )(post)

    x_hat = pl.pallas_call(
        _decode_body,
        grid=(N // RD, S // CD),
        in_specs=[
            pl.BlockSpec((RD, CD), lambda i, j: (i, j)),
            pl.BlockSpec((D, CD), lambda i, j: (0, j)),
            pl.BlockSpec((RD, 1), lambda i, j: (i, 0)),
            pl.BlockSpec((1, D), lambda i, j: (0, 0)),
        ],
        out_specs=pl.BlockSpec((RD, D), lambda i, j: (i, 0)),
        out_shape=jax.ShapeDtypeStruct((N, D), jnp.float32),
    )(post, W_dec, t.reshape(N, 1), b_dec.reshape(1, D))
    return x_hat


# speedup vs baseline: 1.2089x; 1.2089x over previous
import functools

import jax
import jax.numpy as jnp
from jax import lax
from jax.experimental import pallas as pl
from jax.experimental.pallas import tpu as pltpu
from jax.experimental.pallas import tpu_sc as plsc

K = 64
NWORKERS = 32
HSTRIDE = 257


def _encode_body(xb, we, be, bd, out):
    j = pl.program_id(1)
    C = we.shape[0]
    xc = xb[...] - bd[...]
    acc = lax.dot_general(xc, we[...], (((1,), (1,)), ((), ())),
                          preferred_element_type=jnp.float32)
    out[...] = jnp.maximum(acc + be[0, pl.ds(j * C, C)][None, :], 0.0)


def _process_row(buf, r, hist, cnts, sufs, cand, tout, lanes, l257):
    ones_f = jnp.ones((16,), jnp.float32)
    zeros16 = jnp.zeros((16,), jnp.int32)
    zeros16_f = jnp.zeros((16,), jnp.float32)
    true16 = jnp.full((16,), True)
    nvec = buf.shape[0] // 16

    def zero_body(i, _):
        hist[pl.ds(i * 16, 16)] = zeros16_f
        return 0
    lax.fori_loop(0, HSTRIDE, zero_body, 0, unroll=8)

    def hist_body(i, _):
        x = buf[pl.ds(i * 16, 16)]
        b = jnp.maximum(lax.bitcast_convert_type(x, jnp.int32) >> 23, 0)
        plsc.addupdate_scatter(hist, [l257 + b], ones_f, mask=true16)
        return 0
    lax.fori_loop(0, nvec, hist_body, 0, unroll=8)

    def merge_body(i, _):
        acc = zeros16_f
        for s in range(16):
            acc = acc + hist[pl.ds(s * HSTRIDE + i * 16, 16)]
        cnts[pl.ds(i * 16, 16)] = acc
        return 0
    lax.fori_loop(0, 16, merge_body, 0)

    def scan_body(k, carry):
        carrysum, bstar = carry
        i = 15 - k
        cvec = cnts[pl.ds(i * 16, 16)]
        cs = plsc.cumsum(cvec)
        tot = jnp.sum(cvec)
        suffix = (carrysum + tot) - cs + cvec
        sufs[pl.ds(i * 16, 16)] = suffix
        bidx = i * 16 + lanes
        bstar = jnp.maximum(bstar, jnp.where(suffix >= float(K), bidx, -1))
        return (carrysum + tot, bstar)
    _, bstar_vec = lax.fori_loop(
        0, 16, scan_body,
        (jnp.float32(0), jnp.full((16,), -1, jnp.int32)))
    bs = jnp.max(bstar_vec)
    bs_v = zeros16 + bs
    cnt_b = jnp.max(plsc.load_gather(cnts, [bs_v]))
    suf_b = jnp.max(plsc.load_gather(sufs, [bs_v]))
    k2 = (jnp.float32(K) - (suf_b - cnt_b)).astype(jnp.int32)

    def compact_body(i, off_v):
        x = buf[pl.ds(i * 16, 16)]
        bits = lax.bitcast_convert_type(x, jnp.int32)
        b = jnp.maximum(bits >> 23, 0)
        m = b == bs
        pos = plsc.cumsum(m.astype(jnp.int32))
        plsc.store_scatter(cand, [off_v + pos - 1], bits, mask=m)
        return off_v + plsc.all_reduce_population_count(m)
    ncand_v = lax.fori_loop(0, nvec, compact_body, zeros16, unroll=4)
    ncand = jnp.max(ncand_v)
    nv = (ncand + 15) // 16

    def bit_step(bi, t):
        candt = t | (jnp.int32(1) << (22 - bi))

        def inner(v, accv):
            cb = cand[pl.ds(v * 16, 16)]
            valid = (v * 16 + lanes) < ncand
            ok = jnp.logical_and(cb >= candt, valid)
            return accv + jnp.where(ok, 1, 0)
        accv = lax.fori_loop(0, nv, inner, zeros16)
        return jnp.where(jnp.sum(accv) >= k2, candt, t)
    t = lax.fori_loop(0, 23, bit_step, bs << 23)

    tv = lax.bitcast_convert_type(zeros16 + t, jnp.float32)
    plsc.store_scatter(tout, [zeros16 + r], tv, mask=lanes == 0)


def _threshold_body(rows_per_w, post_hbm, t_hbm,
                    buf0, buf1, hist, cnts, sufs, cand, tout, sem):
    c = lax.axis_index("c")
    s = lax.axis_index("s")
    wid = s * 2 + c
    base = wid * rows_per_w
    lanes = lax.iota(jnp.int32, 16)
    l257 = lanes * HSTRIDE
    proc = functools.partial(_process_row, hist=hist, cnts=cnts, sufs=sufs,
                             cand=cand, tout=tout, lanes=lanes, l257=l257)

    pltpu.sync_copy(post_hbm.at[base], buf0)

    def pair_body(p, carry):
        r0 = 2 * p
        h1 = pltpu.async_copy(post_hbm.at[base + r0 + 1], buf1, sem)
        proc(buf0, r0)
        h1.wait()
        nxt = jnp.minimum(r0 + 2, rows_per_w - 1)
        h0 = pltpu.async_copy(post_hbm.at[base + nxt], buf0, sem)
        proc(buf1, r0 + 1)
        h0.wait()
        return carry
    lax.fori_loop(0, rows_per_w // 2, pair_body, 0)

    pltpu.sync_copy(tout, t_hbm.at[pl.ds(base, rows_per_w)])


def _decode_body(xe, wd, tb, bd, out):
    j = pl.program_id(1)

    @pl.when(j == 0)
    def _():
        out[...] = jnp.broadcast_to(bd[...], out.shape)

    v = xe[...]
    m = jnp.where(v >= tb[...], v, 0.0)
    out[...] += lax.dot_general(m, wd[...], (((1,), (1,)), ((), ())),
                                preferred_element_type=jnp.float32)


def kernel(x, W_enc, b_enc, W_dec, b_dec):
    N, D = x.shape
    S = W_enc.shape[0]
    RE = min(1024, N)
    CE = min(1024, S)
    RD = min(512, N)
    CD = min(2048, S)

    post = pl.pallas_call(
        _encode_body,
        grid=(N // RE, S // CE),
        in_specs=[
            pl.BlockSpec((RE, D), lambda i, j: (i, 0)),
            pl.BlockSpec((CE, D), lambda i, j: (j, 0)),
            pl.BlockSpec((1, S), lambda i, j: (0, 0)),
            pl.BlockSpec((1, D), lambda i, j: (0, 0)),
        ],
        out_specs=pl.BlockSpec((RE, CE), lambda i, j: (i, j)),
        out_shape=jax.ShapeDtypeStruct((N, S), jnp.float32),
    )(x, W_enc, b_enc.reshape(1, S), b_dec.reshape(1, D))

    rows_per_w = N // NWORKERS
    mesh = plsc.VectorSubcoreMesh(core_axis_name="c", subcore_axis_name="s",
                                  num_cores=2, num_subcores=16)
    t = pl.kernel(
        functools.partial(_threshold_body, rows_per_w),
        out_type=jax.ShapeDtypeStruct((N,), jnp.float32),
        mesh=mesh,
        compiler_params=pltpu.CompilerParams(needs_layout_passes=False),
        scratch_types=[
            pltpu.VMEM((S,), jnp.float32),
            pltpu.VMEM((S,), jnp.float32),
            pltpu.VMEM((16 * HSTRIDE,), jnp.float32),
            pltpu.VMEM((256,), jnp.float32),
            pltpu.VMEM((256,), jnp.float32),
            pltpu.VMEM((S,), jnp.int32),
            pltpu.VMEM((rows_per_w,), jnp.float32),
            pltpu.SemaphoreType.DMA,
        ],
    )(post)

    x_hat = pl.pallas_call(
        _decode_body,
        grid=(N // RD, S // CD),
        in_specs=[
            pl.BlockSpec((RD, CD), lambda i, j: (i, j)),
            pl.BlockSpec((D, CD), lambda i, j: (0, j)),
            pl.BlockSpec((RD, 1), lambda i, j: (i, 0)),
            pl.BlockSpec((1, D), lambda i, j: (0, 0)),
        ],
        out_specs=pl.BlockSpec((RD, D), lambda i, j: (i, 0)),
        out_shape=jax.ShapeDtypeStruct((N, D), jnp.float32),
    )(post, W_dec, t.reshape(N, 1), b_dec.reshape(1, D))
    return x_hat


# speedup vs baseline: 2.7090x; 2.2410x over previous
import functools

import jax
import jax.numpy as jnp
from jax import lax
from jax.experimental import pallas as pl
from jax.experimental.pallas import tpu as pltpu
from jax.experimental.pallas import tpu_sc as plsc

K = 64
NWORKERS = 32
HSTRIDE = 257


def _encode_body(xb, we, be, bd, out):
    j = pl.program_id(1)
    C = we.shape[0]
    xc = xb[...] - bd[...]
    acc = lax.dot_general(xc, we[...], (((1,), (1,)), ((), ())),
                          preferred_element_type=jnp.float32)
    out[...] = jnp.maximum(acc + be[0, pl.ds(j * C, C)][None, :], 0.0)


def _process_row(buf, r, hist, cnts, sufs, cand, tout, lanes, l257):
    ones_f = jnp.ones((16,), jnp.float32)
    zeros16 = jnp.zeros((16,), jnp.int32)
    zeros16_f = jnp.zeros((16,), jnp.float32)
    true16 = jnp.full((16,), True)
    nvec = buf.shape[0] // 16

    @plsc.parallel_loop(0, 16 * HSTRIDE, 16, unroll=8)
    def _(i):
        hist[pl.ds(i, 16)] = zeros16_f

    @plsc.parallel_loop(0, nvec * 16, 16, unroll=8)
    def _(i):
        x = buf[pl.ds(i, 16)]
        b = jnp.maximum(lax.bitcast_convert_type(x, jnp.int32) >> 23, 0)
        plsc.addupdate_scatter(hist, [l257 + b], ones_f, mask=true16)

    @plsc.parallel_loop(0, 256, 16, unroll=2)
    def _(i):
        acc = zeros16_f
        for s in range(16):
            acc = acc + hist[pl.ds(s * HSTRIDE + i, 16)]
        cnts[pl.ds(i, 16)] = acc

    def scan_body(k, carry):
        carrysum, bstar = carry
        i = 15 - k
        cvec = cnts[pl.ds(i * 16, 16)]
        cs = plsc.cumsum(cvec)
        tot = jnp.sum(cvec)
        suffix = (carrysum + tot) - cs + cvec
        sufs[pl.ds(i * 16, 16)] = suffix
        bidx = i * 16 + lanes
        bstar = jnp.maximum(bstar, jnp.where(suffix >= float(K), bidx, -1))
        return (carrysum + tot, bstar)
    _, bstar_vec = lax.fori_loop(
        0, 16, scan_body,
        (jnp.float32(0), jnp.full((16,), -1, jnp.int32)))
    bs = jnp.max(bstar_vec)
    bs_v = zeros16 + bs
    cnt_b = jnp.max(plsc.load_gather(cnts, [bs_v]))
    suf_b = jnp.max(plsc.load_gather(sufs, [bs_v]))
    k2 = (jnp.float32(K) - (suf_b - cnt_b)).astype(jnp.int32)

    @plsc.parallel_loop(0, nvec * 16, 16, unroll=4, carry=zeros16)
    def ncand_v(i, off_v):
        x = buf[pl.ds(i, 16)]
        bits = lax.bitcast_convert_type(x, jnp.int32)
        b = jnp.maximum(bits >> 23, 0)
        m = b == bs
        pos = plsc.cumsum(m.astype(jnp.int32))
        plsc.store_scatter(cand, [off_v + pos - 1], bits, mask=m)
        return off_v + plsc.all_reduce_population_count(m)
    ncand = jnp.max(ncand_v)
    nv16 = ((ncand + 15) // 16) * 16

    def bit_step(bi, t):
        candt = t | (jnp.int32(1) << (22 - bi))

        @plsc.parallel_loop(0, nv16, 16, carry=zeros16)
        def accv(v, accv):
            cb = cand[pl.ds(v, 16)]
            valid = (v + lanes) < ncand
            ok = jnp.logical_and(cb >= candt, valid)
            return accv + jnp.where(ok, 1, 0)
        return jnp.where(jnp.sum(accv) >= k2, candt, t)
    t = lax.fori_loop(0, 23, bit_step, bs << 23)

    tv = lax.bitcast_convert_type(zeros16 + t, jnp.float32)
    plsc.store_scatter(tout, [zeros16 + r], tv, mask=lanes == 0)


def _threshold_body(rows_per_w, post_hbm, t_hbm,
                    buf0, buf1, hist, cnts, sufs, cand, tout, sem):
    c = lax.axis_index("c")
    s = lax.axis_index("s")
    wid = s * 2 + c
    base = wid * rows_per_w
    lanes = lax.iota(jnp.int32, 16)
    l257 = lanes * HSTRIDE
    proc = functools.partial(_process_row, hist=hist, cnts=cnts, sufs=sufs,
                             cand=cand, tout=tout, lanes=lanes, l257=l257)

    pltpu.sync_copy(post_hbm.at[base], buf0)

    def pair_body(p, carry):
        r0 = 2 * p
        h1 = pltpu.async_copy(post_hbm.at[base + r0 + 1], buf1, sem)
        proc(buf0, r0)
        h1.wait()
        nxt = jnp.minimum(r0 + 2, rows_per_w - 1)
        h0 = pltpu.async_copy(post_hbm.at[base + nxt], buf0, sem)
        proc(buf1, r0 + 1)
        h0.wait()
        return carry
    lax.fori_loop(0, rows_per_w // 2, pair_body, 0)

    pltpu.sync_copy(tout, t_hbm.at[pl.ds(base, rows_per_w)])


def _decode_body(xe, wd, tb, bd, out):
    j = pl.program_id(1)

    @pl.when(j == 0)
    def _():
        out[...] = jnp.broadcast_to(bd[...], out.shape)

    v = xe[...]
    m = jnp.where(v >= tb[...], v, 0.0)
    out[...] += lax.dot_general(m, wd[...], (((1,), (1,)), ((), ())),
                                preferred_element_type=jnp.float32)


def kernel(x, W_enc, b_enc, W_dec, b_dec):
    N, D = x.shape
    S = W_enc.shape[0]
    RE = min(1024, N)
    CE = min(1024, S)
    RD = min(512, N)
    CD = min(2048, S)

    post = pl.pallas_call(
        _encode_body,
        grid=(N // RE, S // CE),
        in_specs=[
            pl.BlockSpec((RE, D), lambda i, j: (i, 0)),
            pl.BlockSpec((CE, D), lambda i, j: (j, 0)),
            pl.BlockSpec((1, S), lambda i, j: (0, 0)),
            pl.BlockSpec((1, D), lambda i, j: (0, 0)),
        ],
        out_specs=pl.BlockSpec((RE, CE), lambda i, j: (i, j)),
        out_shape=jax.ShapeDtypeStruct((N, S), jnp.float32),
    )(x, W_enc, b_enc.reshape(1, S), b_dec.reshape(1, D))

    rows_per_w = N // NWORKERS
    mesh = plsc.VectorSubcoreMesh(core_axis_name="c", subcore_axis_name="s",
                                  num_cores=2, num_subcores=16)
    t = pl.kernel(
        functools.partial(_threshold_body, rows_per_w),
        out_type=jax.ShapeDtypeStruct((N,), jnp.float32),
        mesh=mesh,
        compiler_params=pltpu.CompilerParams(needs_layout_passes=False),
        scratch_types=[
            pltpu.VMEM((S,), jnp.float32),
            pltpu.VMEM((S,), jnp.float32),
            pltpu.VMEM((16 * HSTRIDE,), jnp.float32),
            pltpu.VMEM((256,), jnp.float32),
            pltpu.VMEM((256,), jnp.float32),
            pltpu.VMEM((S,), jnp.int32),
            pltpu.VMEM((rows_per_w,), jnp.float32),
            pltpu.SemaphoreType.DMA,
        ],
    )(post)

    x_hat = pl.pallas_call(
        _decode_body,
        grid=(N // RD, S // CD),
        in_specs=[
            pl.BlockSpec((RD, CD), lambda i, j: (i, j)),
            pl.BlockSpec((D, CD), lambda i, j: (0, j)),
            pl.BlockSpec((RD, 1), lambda i, j: (i, 0)),
            pl.BlockSpec((1, D), lambda i, j: (0, 0)),
        ],
        out_specs=pl.BlockSpec((RD, D), lambda i, j: (i, 0)),
        out_shape=jax.ShapeDtypeStruct((N, D), jnp.float32),
    )(post, W_dec, t.reshape(N, 1), b_dec.reshape(1, D))
    return x_hat


# speedup vs baseline: 2.9800x; 1.1000x over previous
import functools

import jax
import jax.numpy as jnp
from jax import lax
from jax.experimental import pallas as pl
from jax.experimental.pallas import tpu as pltpu
from jax.experimental.pallas import tpu_sc as plsc

K = 64
NWORKERS = 32
HSTRIDE = 257


def _encode_body(xb, we, be, bd, out):
    j = pl.program_id(1)
    C = we.shape[0]
    xc = xb[...] - bd[...]
    acc = lax.dot_general(xc, we[...], (((1,), (1,)), ((), ())),
                          preferred_element_type=jnp.float32)
    out[...] = jnp.maximum(acc + be[0, pl.ds(j * C, C)][None, :], 0.0)


def _process_row(buf, r, hist, cnts, sufs, cand, tout, lanes, l257):
    ones_f = jnp.ones((16,), jnp.float32)
    zeros16 = jnp.zeros((16,), jnp.int32)
    zeros16_f = jnp.zeros((16,), jnp.float32)
    true16 = jnp.full((16,), True)
    nvec = buf.shape[0] // 16

    @plsc.parallel_loop(0, 16 * HSTRIDE, 16, unroll=8)
    def _(i):
        hist[pl.ds(i, 16)] = zeros16_f

    @plsc.parallel_loop(0, nvec * 16, 16, unroll=8)
    def _(i):
        x = buf[pl.ds(i, 16)]
        b = jnp.maximum(lax.bitcast_convert_type(x, jnp.int32) >> 23, 0)
        plsc.addupdate_scatter(hist, [l257 + b], ones_f, mask=true16)

    @plsc.parallel_loop(0, 256, 16, unroll=2)
    def _(i):
        acc = zeros16_f
        for s in range(16):
            acc = acc + hist[pl.ds(s * HSTRIDE + i, 16)]
        cnts[pl.ds(i, 16)] = acc

    def scan_body(k, carry):
        carrysum, bstar = carry
        i = 15 - k
        cvec = cnts[pl.ds(i * 16, 16)]
        cs = plsc.cumsum(cvec)
        tot = jnp.sum(cvec)
        suffix = (carrysum + tot) - cs + cvec
        sufs[pl.ds(i * 16, 16)] = suffix
        bidx = i * 16 + lanes
        bstar = jnp.maximum(bstar, jnp.where(suffix >= float(K), bidx, -1))
        return (carrysum + tot, bstar)
    _, bstar_vec = lax.fori_loop(
        0, 16, scan_body,
        (jnp.float32(0), jnp.full((16,), -1, jnp.int32)))
    bs = jnp.max(bstar_vec)
    bs_v = zeros16 + bs
    cnt_b = jnp.max(plsc.load_gather(cnts, [bs_v]))
    suf_b = jnp.max(plsc.load_gather(sufs, [bs_v]))
    k2 = (jnp.float32(K) - (suf_b - cnt_b)).astype(jnp.int32)

    @plsc.parallel_loop(0, nvec * 16, 16, unroll=4, carry=zeros16)
    def ncand_v(i, off_v):
        x = buf[pl.ds(i, 16)]
        bits = lax.bitcast_convert_type(x, jnp.int32)
        b = jnp.maximum(bits >> 23, 0)
        m = b == bs
        pos = plsc.cumsum(m.astype(jnp.int32))
        plsc.store_scatter(cand, [off_v + pos - 1], bits, mask=m)
        return off_v + plsc.all_reduce_population_count(m)
    ncand = jnp.max(ncand_v)
    nv16 = ((ncand + 15) // 16) * 16

    def bit_step(bi, t):
        candt = t | (jnp.int32(1) << (22 - bi))

        @plsc.parallel_loop(0, nv16, 16, carry=zeros16)
        def accv(v, accv):
            cb = cand[pl.ds(v, 16)]
            valid = (v + lanes) < ncand
            ok = jnp.logical_and(cb >= candt, valid)
            return accv + jnp.where(ok, 1, 0)
        return jnp.where(jnp.sum(accv) >= k2, candt, t)
    t = lax.fori_loop(0, 23, bit_step, bs << 23)

    tv = lax.bitcast_convert_type(zeros16 + t, jnp.float32)
    plsc.store_scatter(tout, [zeros16 + r], tv, mask=lanes == 0)


def _threshold_body(rows_per_w, post_hbm, t_hbm,
                    buf0, buf1, hist, cnts, sufs, cand, tout, sem):
    c = lax.axis_index("c")
    s = lax.axis_index("s")
    wid = s * 2 + c
    base = wid * rows_per_w
    lanes = lax.iota(jnp.int32, 16)
    l257 = lanes * HSTRIDE
    proc = functools.partial(_process_row, hist=hist, cnts=cnts, sufs=sufs,
                             cand=cand, tout=tout, lanes=lanes, l257=l257)

    pltpu.sync_copy(post_hbm.at[base], buf0)

    def pair_body(p, carry):
        r0 = 2 * p
        h1 = pltpu.async_copy(post_hbm.at[base + r0 + 1], buf1, sem)
        proc(buf0, r0)
        h1.wait()
        nxt = jnp.minimum(r0 + 2, rows_per_w - 1)
        h0 = pltpu.async_copy(post_hbm.at[base + nxt], buf0, sem)
        proc(buf1, r0 + 1)
        h0.wait()
        return carry
    lax.fori_loop(0, rows_per_w // 2, pair_body, 0)

    pltpu.sync_copy(tout, t_hbm.at[pl.ds(base, rows_per_w)])


def _decode_body(xe, wd, tb, bd, out):
    j = pl.program_id(1)

    @pl.when(j == 0)
    def _():
        out[...] = jnp.broadcast_to(bd[...], out.shape)

    v = xe[...]
    m = jnp.where(v >= tb[...], v, 0.0).astype(jnp.bfloat16)
    out[...] += lax.dot_general(m, wd[...], (((1,), (1,)), ((), ())),
                                preferred_element_type=jnp.float32)


def _forward(x, W_enc, b_enc, W_dec_bf, b_dec):
    N, D = x.shape
    S = W_enc.shape[0]
    RE = min(1024, N)
    CE = min(1024, S)
    RD = min(512, N)
    CD = min(2048, S)

    post = pl.pallas_call(
        _encode_body,
        grid=(N // RE, S // CE),
        in_specs=[
            pl.BlockSpec((RE, D), lambda i, j: (i, 0)),
            pl.BlockSpec((CE, D), lambda i, j: (j, 0)),
            pl.BlockSpec((1, S), lambda i, j: (0, 0)),
            pl.BlockSpec((1, D), lambda i, j: (0, 0)),
        ],
        out_specs=pl.BlockSpec((RE, CE), lambda i, j: (i, j)),
        out_shape=jax.ShapeDtypeStruct((N, S), jnp.float32),
    )(x, W_enc, b_enc.reshape(1, S), b_dec.reshape(1, D))

    rows_per_w = N // NWORKERS
    mesh = plsc.VectorSubcoreMesh(core_axis_name="c", subcore_axis_name="s",
                                  num_cores=2, num_subcores=16)
    t = pl.kernel(
        functools.partial(_threshold_body, rows_per_w),
        out_type=jax.ShapeDtypeStruct((N,), jnp.float32),
        mesh=mesh,
        compiler_params=pltpu.CompilerParams(needs_layout_passes=False),
        scratch_types=[
            pltpu.VMEM((S,), jnp.float32),
            pltpu.VMEM((S,), jnp.float32),
            pltpu.VMEM((16 * HSTRIDE,), jnp.float32),
            pltpu.VMEM((256,), jnp.float32),
            pltpu.VMEM((256,), jnp.float32),
            pltpu.VMEM((S,), jnp.int32),
            pltpu.VMEM((rows_per_w,), jnp.float32),
            pltpu.SemaphoreType.DMA,
        ],
    )(post)

    x_hat = pl.pallas_call(
        _decode_body,
        grid=(N // RD, S // CD),
        in_specs=[
            pl.BlockSpec((RD, CD), lambda i, j: (i, j)),
            pl.BlockSpec((D, CD), lambda i, j: (0, j)),
            pl.BlockSpec((RD, 1), lambda i, j: (i, 0)),
            pl.BlockSpec((1, D), lambda i, j: (0, 0)),
        ],
        out_specs=pl.BlockSpec((RD, D), lambda i, j: (i, 0)),
        out_shape=jax.ShapeDtypeStruct((N, D), jnp.float32),
    )(post, W_dec_bf, t.reshape(N, 1), b_dec.reshape(1, D))
    return x_hat


def kernel(x, W_enc, b_enc, W_dec, b_dec):
    N = x.shape[0]
    W_dec_bf = W_dec.astype(jnp.bfloat16)
    if N % (2 * NWORKERS * 2) == 0:
        h = N // 2
        y0 = _forward(x[:h], W_enc, b_enc, W_dec_bf, b_dec)
        y1 = _forward(x[h:], W_enc, b_enc, W_dec_bf, b_dec)
        return jnp.concatenate([y0, y1], axis=0)
    return _forward(x, W_enc, b_enc, W_dec_bf, b_dec)


# speedup vs baseline: 3.5701x; 1.1980x over previous
import functools

import jax
import jax.numpy as jnp
from jax import lax
from jax.experimental import pallas as pl
from jax.experimental.pallas import tpu as pltpu
from jax.experimental.pallas import tpu_sc as plsc

K = 64
NWORKERS = 32
HSTRIDE = 257


def _encode_body(xb, we, be, bd, out):
    j = pl.program_id(1)
    C = we.shape[0]
    xc = xb[...] - bd[...]
    acc = lax.dot_general(xc, we[...], (((1,), (1,)), ((), ())),
                          preferred_element_type=jnp.float32)
    out[...] = jnp.maximum(acc + be[0, pl.ds(j * C, C)][None, :], 0.0)


def _process_row(buf, r, hist, cnts, sufs, cand, tout, lanes, l257):
    ones_f = jnp.ones((16,), jnp.float32)
    zeros16 = jnp.zeros((16,), jnp.int32)
    zeros16_f = jnp.zeros((16,), jnp.float32)
    true16 = jnp.full((16,), True)
    nvec = buf.shape[0] // 16

    @plsc.parallel_loop(0, 16 * HSTRIDE, 16, unroll=8)
    def _(i):
        hist[pl.ds(i, 16)] = zeros16_f

    @plsc.parallel_loop(0, nvec * 16, 16, unroll=8)
    def _(i):
        x = buf[pl.ds(i, 16)]
        b = jnp.maximum(lax.bitcast_convert_type(x, jnp.int32) >> 23, 0)
        plsc.addupdate_scatter(hist, [l257 + b], ones_f, mask=true16)

    @plsc.parallel_loop(0, 256, 16, unroll=2)
    def _(i):
        acc = zeros16_f
        for s in range(16):
            acc = acc + hist[pl.ds(s * HSTRIDE + i, 16)]
        cnts[pl.ds(i, 16)] = acc

    def scan_body(k, carry):
        carrysum, bstar = carry
        i = 15 - k
        cvec = cnts[pl.ds(i * 16, 16)]
        cs = plsc.cumsum(cvec)
        tot = jnp.sum(cvec)
        suffix = (carrysum + tot) - cs + cvec
        sufs[pl.ds(i * 16, 16)] = suffix
        bidx = i * 16 + lanes
        bstar = jnp.maximum(bstar, jnp.where(suffix >= float(K), bidx, -1))
        return (carrysum + tot, bstar)
    _, bstar_vec = lax.fori_loop(
        0, 16, scan_body,
        (jnp.float32(0), jnp.full((16,), -1, jnp.int32)))
    bs = jnp.max(bstar_vec)
    bs_v = zeros16 + bs
    cnt_b = jnp.max(plsc.load_gather(cnts, [bs_v]))
    suf_b = jnp.max(plsc.load_gather(sufs, [bs_v]))
    k2 = (jnp.float32(K) - (suf_b - cnt_b)).astype(jnp.int32)

    @plsc.parallel_loop(0, nvec * 16, 16, unroll=4, carry=zeros16)
    def ncand_v(i, off_v):
        x = buf[pl.ds(i, 16)]
        bits = lax.bitcast_convert_type(x, jnp.int32)
        b = jnp.maximum(bits >> 23, 0)
        m = b == bs
        pos = plsc.cumsum(m.astype(jnp.int32))
        plsc.store_scatter(cand, [off_v + pos - 1], bits, mask=m)
        return off_v + plsc.all_reduce_population_count(m)
    ncand = jnp.max(ncand_v)
    nv16 = ((ncand + 15) // 16) * 16

    def bit_step(bi, t):
        candt = t | (jnp.int32(1) << (22 - bi))

        @plsc.parallel_loop(0, nv16, 16, carry=zeros16)
        def accv(v, accv):
            cb = cand[pl.ds(v, 16)]
            valid = (v + lanes) < ncand
            ok = jnp.logical_and(cb >= candt, valid)
            return accv + jnp.where(ok, 1, 0)
        return jnp.where(jnp.sum(accv) >= k2, candt, t)
    t = lax.fori_loop(0, 23, bit_step, bs << 23)

    tv = lax.bitcast_convert_type(zeros16 + t, jnp.float32)
    plsc.store_scatter(tout, [zeros16 + r], tv, mask=lanes == 0)


def _threshold_body(rows_per_w, post_hbm, t_hbm,
                    buf0, buf1, hist, cnts, sufs, cand, tout, sem):
    c = lax.axis_index("c")
    s = lax.axis_index("s")
    wid = s * 2 + c
    base = wid * rows_per_w
    lanes = lax.iota(jnp.int32, 16)
    l257 = lanes * HSTRIDE
    proc = functools.partial(_process_row, hist=hist, cnts=cnts, sufs=sufs,
                             cand=cand, tout=tout, lanes=lanes, l257=l257)

    pltpu.sync_copy(post_hbm.at[base], buf0)

    def pair_body(p, carry):
        r0 = 2 * p
        h1 = pltpu.async_copy(post_hbm.at[base + r0 + 1], buf1, sem)
        proc(buf0, r0)
        h1.wait()
        nxt = jnp.minimum(r0 + 2, rows_per_w - 1)
        h0 = pltpu.async_copy(post_hbm.at[base + nxt], buf0, sem)
        proc(buf1, r0 + 1)
        h0.wait()
        return carry
    lax.fori_loop(0, rows_per_w // 2, pair_body, 0)

    pltpu.sync_copy(tout, t_hbm.at[pl.ds(base, rows_per_w)])


def _decode_body(xe, wd, tb, bd, out):
    j = pl.program_id(1)

    @pl.when(j == 0)
    def _():
        out[...] = jnp.broadcast_to(bd[...], out.shape)

    v = xe[...]
    m = jnp.where(v >= tb[...], v, 0.0).astype(jnp.bfloat16)
    out[...] += lax.dot_general(m, wd[...], (((1,), (1,)), ((), ())),
                                preferred_element_type=jnp.float32)


def _forward(x, W_enc, b_enc, W_dec_bf, b_dec):
    N, D = x.shape
    S = W_enc.shape[0]
    RE = min(1024, N)
    CE = min(1024, S)
    RD = min(512, N)
    CD = min(2048, S)

    post = pl.pallas_call(
        _encode_body,
        grid=(N // RE, S // CE),
        in_specs=[
            pl.BlockSpec((RE, D), lambda i, j: (i, 0)),
            pl.BlockSpec((CE, D), lambda i, j: (j, 0)),
            pl.BlockSpec((1, S), lambda i, j: (0, 0)),
            pl.BlockSpec((1, D), lambda i, j: (0, 0)),
        ],
        out_specs=pl.BlockSpec((RE, CE), lambda i, j: (i, j)),
        out_shape=jax.ShapeDtypeStruct((N, S), jnp.float32),
    )(x, W_enc, b_enc.reshape(1, S), b_dec.reshape(1, D))

    rows_per_w = N // NWORKERS
    mesh = plsc.VectorSubcoreMesh(core_axis_name="c", subcore_axis_name="s",
                                  num_cores=2, num_subcores=16)
    t = pl.kernel(
        functools.partial(_threshold_body, rows_per_w),
        out_type=jax.ShapeDtypeStruct((N,), jnp.float32),
        mesh=mesh,
        compiler_params=pltpu.CompilerParams(needs_layout_passes=False),
        scratch_types=[
            pltpu.VMEM((S,), jnp.float32),
            pltpu.VMEM((S,), jnp.float32),
            pltpu.VMEM((16 * HSTRIDE,), jnp.float32),
            pltpu.VMEM((256,), jnp.float32),
            pltpu.VMEM((256,), jnp.float32),
            pltpu.VMEM((S,), jnp.int32),
            pltpu.VMEM((rows_per_w,), jnp.float32),
            pltpu.SemaphoreType.DMA,
        ],
    )(post)

    x_hat = pl.pallas_call(
        _decode_body,
        grid=(N // RD, S // CD),
        in_specs=[
            pl.BlockSpec((RD, CD), lambda i, j: (i, j)),
            pl.BlockSpec((D, CD), lambda i, j: (0, j)),
            pl.BlockSpec((RD, 1), lambda i, j: (i, 0)),
            pl.BlockSpec((1, D), lambda i, j: (0, 0)),
        ],
        out_specs=pl.BlockSpec((RD, D), lambda i, j: (i, 0)),
        out_shape=jax.ShapeDtypeStruct((N, D), jnp.float32),
    )(post, W_dec_bf, t.reshape(N, 1), b_dec.reshape(1, D))
    return x_hat


def kernel(x, W_enc, b_enc, W_dec, b_dec):
    N = x.shape[0]
    W_dec_bf = W_dec.astype(jnp.bfloat16)
    nsplit = 4 if N % (4 * NWORKERS * 2) == 0 else 1
    if nsplit > 1:
        h = N // nsplit
        ys = [_forward(x[i * h:(i + 1) * h], W_enc, b_enc, W_dec_bf, b_dec)
              for i in range(nsplit)]
        return jnp.concatenate(ys, axis=0)
    return _forward(x, W_enc, b_enc, W_dec_bf, b_dec)


# speedup vs baseline: 3.6280x; 1.0162x over previous
import functools

import jax
import jax.numpy as jnp
from jax import lax
from jax.experimental import pallas as pl
from jax.experimental.pallas import tpu as pltpu
from jax.experimental.pallas import tpu_sc as plsc

K = 64
NWORKERS = 32
HSTRIDE = 257


def _encode_body(xb, we, be, bd, out):
    j = pl.program_id(1)
    C = we.shape[0]
    xc = xb[...] - bd[...]
    acc = lax.dot_general(xc, we[...], (((1,), (1,)), ((), ())),
                          preferred_element_type=jnp.float32)
    out[...] = jnp.maximum(acc + be[0, pl.ds(j * C, C)][None, :], 0.0)


def _process_row(buf, r, hist, cnts, sufs, cand, tout, lanes, l257):
    ones_f = jnp.ones((16,), jnp.float32)
    zeros16 = jnp.zeros((16,), jnp.int32)
    zeros16_f = jnp.zeros((16,), jnp.float32)
    true16 = jnp.full((16,), True)
    nvec = buf.shape[0] // 16

    @plsc.parallel_loop(0, 16 * HSTRIDE, 16, unroll=8)
    def _(i):
        hist[pl.ds(i, 16)] = zeros16_f

    @plsc.parallel_loop(0, nvec * 16, 16, unroll=8)
    def _(i):
        x = buf[pl.ds(i, 16)]
        b = jnp.maximum(lax.bitcast_convert_type(x, jnp.int32) >> 23, 0)
        plsc.addupdate_scatter(hist, [l257 + b], ones_f, mask=true16)

    @plsc.parallel_loop(0, 256, 16, unroll=2)
    def _(i):
        acc = zeros16_f
        for s in range(16):
            acc = acc + hist[pl.ds(s * HSTRIDE + i, 16)]
        cnts[pl.ds(i, 16)] = acc

    def scan_body(k, carry):
        carrysum, bstar = carry
        i = 15 - k
        cvec = cnts[pl.ds(i * 16, 16)]
        cs = plsc.cumsum(cvec)
        tot = jnp.sum(cvec)
        suffix = (carrysum + tot) - cs + cvec
        sufs[pl.ds(i * 16, 16)] = suffix
        bidx = i * 16 + lanes
        bstar = jnp.maximum(bstar, jnp.where(suffix >= float(K), bidx, -1))
        return (carrysum + tot, bstar)
    _, bstar_vec = lax.fori_loop(
        0, 16, scan_body,
        (jnp.float32(0), jnp.full((16,), -1, jnp.int32)))
    bs = jnp.max(bstar_vec)
    bs_v = zeros16 + bs
    cnt_b = jnp.max(plsc.load_gather(cnts, [bs_v]))
    suf_b = jnp.max(plsc.load_gather(sufs, [bs_v]))
    k2 = (jnp.float32(K) - (suf_b - cnt_b)).astype(jnp.int32)

    @plsc.parallel_loop(0, nvec * 16, 16, unroll=4, carry=zeros16)
    def ncand_v(i, off_v):
        x = buf[pl.ds(i, 16)]
        bits = lax.bitcast_convert_type(x, jnp.int32)
        b = jnp.maximum(bits >> 23, 0)
        m = b == bs
        pos = plsc.cumsum(m.astype(jnp.int32))
        plsc.store_scatter(cand, [off_v + pos - 1], bits, mask=m)
        return off_v + plsc.all_reduce_population_count(m)
    ncand = jnp.max(ncand_v)
    nv16 = ((ncand + 15) // 16) * 16

    def bit_step(bi, t):
        candt = t | (jnp.int32(1) << (22 - bi))

        @plsc.parallel_loop(0, nv16, 16, carry=zeros16)
        def accv(v, accv):
            cb = cand[pl.ds(v, 16)]
            valid = (v + lanes) < ncand
            ok = jnp.logical_and(cb >= candt, valid)
            return accv + jnp.where(ok, 1, 0)
        return jnp.where(jnp.sum(accv) >= k2, candt, t)
    t = lax.fori_loop(0, 23, bit_step, bs << 23)

    tv = lax.bitcast_convert_type(zeros16 + t, jnp.float32)
    plsc.store_scatter(tout, [zeros16 + r], tv, mask=lanes == 0)


def _threshold_body(rows_per_w, post_hbm, t_hbm,
                    buf0, buf1, hist, cnts, sufs, cand, tout, sem):
    c = lax.axis_index("c")
    s = lax.axis_index("s")
    wid = s * 2 + c
    base = wid * rows_per_w
    lanes = lax.iota(jnp.int32, 16)
    l257 = lanes * HSTRIDE
    proc = functools.partial(_process_row, hist=hist, cnts=cnts, sufs=sufs,
                             cand=cand, tout=tout, lanes=lanes, l257=l257)

    pltpu.sync_copy(post_hbm.at[base], buf0)

    def pair_body(p, carry):
        r0 = 2 * p
        h1 = pltpu.async_copy(post_hbm.at[base + r0 + 1], buf1, sem)
        proc(buf0, r0)
        h1.wait()
        nxt = jnp.minimum(r0 + 2, rows_per_w - 1)
        h0 = pltpu.async_copy(post_hbm.at[base + nxt], buf0, sem)
        proc(buf1, r0 + 1)
        h0.wait()
        return carry
    lax.fori_loop(0, rows_per_w // 2, pair_body, 0)

    pltpu.sync_copy(tout, t_hbm.at[pl.ds(base, rows_per_w)])


def _decode_body(xe, wd, tb, bd, out):
    j = pl.program_id(1)

    @pl.when(j == 0)
    def _():
        out[...] = jnp.broadcast_to(bd[...], out.shape)

    v = xe[...]
    m = jnp.where(v >= tb[...], v, 0.0).astype(jnp.bfloat16)
    out[...] += lax.dot_general(m, wd[...], (((1,), (1,)), ((), ())),
                                preferred_element_type=jnp.float32)


def _forward(x, W_enc, b_enc, W_dec_bf, b_dec):
    N, D = x.shape
    S = W_enc.shape[0]
    RE = min(1024, N)
    CE = min(1024, S)
    RD = min(512, N)
    CD = min(2048, S)

    post = pl.pallas_call(
        _encode_body,
        grid=(N // RE, S // CE),
        in_specs=[
            pl.BlockSpec((RE, D), lambda i, j: (i, 0)),
            pl.BlockSpec((CE, D), lambda i, j: (j, 0)),
            pl.BlockSpec((1, S), lambda i, j: (0, 0)),
            pl.BlockSpec((1, D), lambda i, j: (0, 0)),
        ],
        out_specs=pl.BlockSpec((RE, CE), lambda i, j: (i, j)),
        out_shape=jax.ShapeDtypeStruct((N, S), jnp.float32),
    )(x, W_enc, b_enc.reshape(1, S), b_dec.reshape(1, D))

    rows_per_w = N // NWORKERS
    mesh = plsc.VectorSubcoreMesh(core_axis_name="c", subcore_axis_name="s",
                                  num_cores=2, num_subcores=16)
    t = pl.kernel(
        functools.partial(_threshold_body, rows_per_w),
        out_type=jax.ShapeDtypeStruct((N,), jnp.float32),
        mesh=mesh,
        compiler_params=pltpu.CompilerParams(needs_layout_passes=False),
        scratch_types=[
            pltpu.VMEM((S,), jnp.float32),
            pltpu.VMEM((S,), jnp.float32),
            pltpu.VMEM((16 * HSTRIDE,), jnp.float32),
            pltpu.VMEM((256,), jnp.float32),
            pltpu.VMEM((256,), jnp.float32),
            pltpu.VMEM((S,), jnp.int32),
            pltpu.VMEM((rows_per_w,), jnp.float32),
            pltpu.SemaphoreType.DMA,
        ],
    )(post)

    x_hat = pl.pallas_call(
        _decode_body,
        grid=(N // RD, S // CD),
        in_specs=[
            pl.BlockSpec((RD, CD), lambda i, j: (i, j)),
            pl.BlockSpec((D, CD), lambda i, j: (0, j)),
            pl.BlockSpec((RD, 1), lambda i, j: (i, 0)),
            pl.BlockSpec((1, D), lambda i, j: (0, 0)),
        ],
        out_specs=pl.BlockSpec((RD, D), lambda i, j: (i, 0)),
        out_shape=jax.ShapeDtypeStruct((N, D), jnp.float32),
    )(post, W_dec_bf, t.reshape(N, 1), b_dec.reshape(1, D))
    return x_hat


def kernel(x, W_enc, b_enc, W_dec, b_dec):
    N = x.shape[0]
    W_dec_bf = W_dec.astype(jnp.bfloat16)
    nsplit = 8 if N % (8 * NWORKERS * 2) == 0 else 1
    if nsplit > 1:
        h = N // nsplit
        ys = [_forward(x[i * h:(i + 1) * h], W_enc, b_enc, W_dec_bf, b_dec)
              for i in range(nsplit)]
        return jnp.concatenate(ys, axis=0)
    return _forward(x, W_enc, b_enc, W_dec_bf, b_dec)
